# Initial kernel scaffold; baseline (speedup 1.0000x reference)
import jax
import jax.numpy as jnp
from jax.experimental import pallas as pl


def kernel(rays_o, rays_d, near, far, W1, b1, W2, b2):
    raise NotImplementedError("write your pallas kernel here")


# speedup vs baseline: 1.3522x; 1.3522x over previous
import jax
import jax.numpy as jnp
from jax.experimental import pallas as pl

N_SAMPLES = 64
N_IMP = 16
STEPS = 4
HID = 256
R_BLOCK = 128


def _fiota(shape, dim):
    return jax.lax.broadcasted_iota(jnp.int32, shape, dim).astype(jnp.float32)


def _softplus(x):
    return jnp.maximum(x, 0.0) + jnp.log1p(jnp.exp(-jnp.abs(x)))


def _ssampler_kernel(o_ref, d_ref, nf_ref, W1_ref, b1_ref, w2_ref, b2_ref, out_ref):
    f32 = jnp.float32
    R = o_ref.shape[0]
    o = o_ref[...]
    d = d_ref[...]
    W1 = W1_ref[...]
    b1 = b1_ref[...]
    w2 = w2_ref[...]
    b2 = b2_ref[0, 0]
    near = nf_ref[:, 0:1]
    far = nf_ref[:, 1:2]

    a_o = jnp.dot(o, W1, preferred_element_type=f32, precision=jax.lax.Precision.HIGHEST)
    a_d = jnp.dot(d, W1, preferred_element_type=f32, precision=jax.lax.Precision.HIGHEST)
    oo = jnp.sum(o * o, axis=-1, keepdims=True)
    od = jnp.sum(o * d, axis=-1, keepdims=True)
    dd = jnp.sum(d * d, axis=-1, keepdims=True)

    b1b = b1.reshape(1, 1, HID)
    w2b = w2.reshape(1, 1, HID)

    def sdf_eval(zv):
        h = a_o[:, None, :] + zv[:, :, None] * a_d[:, None, :] + b1b
        return jnp.sum(_softplus(h) * w2b, axis=-1) + b2

    t = _fiota((1, N_SAMPLES), 1) * (1.0 / (N_SAMPLES - 1))
    z = near * (1.0 - t) + far * t
    sdf = sdf_eval(z)

    for step in range(STEPS):
        inv_s = float(64 * 2 ** step)
        n = z.shape[1]
        m = n - 1
        last = step == STEPS - 1

        rad2 = oo + 2.0 * z * od + z * z * dd
        inside = jnp.logical_or(rad2[:, :-1] < 1.0, rad2[:, 1:] < 1.0).astype(f32)
        ps, nsdf = sdf[:, :-1], sdf[:, 1:]
        pz, nz = z[:, :-1], z[:, 1:]
        mid = (ps + nsdf) * 0.5
        dist = nz - pz
        cos = (nsdf - ps) / (dist + 1e-5)
        prev_cos = jnp.concatenate([jnp.zeros((R, 1), f32), cos[:, :-1]], axis=1)
        cos = jnp.minimum(prev_cos, cos)
        cos = jnp.clip(cos, -1000.0, 0.0) * inside
        pe = mid - cos * dist * 0.5
        ne = mid + cos * dist * 0.5
        pc = jax.nn.sigmoid(pe * inv_s)
        nc = jax.nn.sigmoid(ne * inv_s)
        alpha = (pc - nc + 1e-5) / (pc + 1e-5)
        lg = jnp.log1p(1e-7 - alpha)
        row = _fiota((m, m), 0)
        col = _fiota((m, m), 1)
        strict_u = (row < col).astype(f32)
        incl_u = (row <= col).astype(f32)
        trans = jnp.exp(jnp.dot(lg, strict_u, preferred_element_type=f32, precision=jax.lax.Precision.HIGHEST))
        w = alpha * trans + 1e-5

        pdf = w / jnp.sum(w, axis=-1, keepdims=True)
        cdf_core = jnp.dot(pdf, incl_u, preferred_element_type=f32, precision=jax.lax.Precision.HIGHEST)
        cdf = jnp.concatenate([jnp.zeros((R, 1), f32), cdf_core], axis=1)
        zl, zr = z[:, :-1], z[:, 1:]
        cl, cr = cdf[:, :-1], cdf[:, 1:]
        denom = cr - cl
        denom = jnp.where(denom < 1e-5, 1.0, denom)
        bb = (zr - zl) / denom
        aa = zl - cl * bb
        z0c = jnp.zeros((R, 1), f32)
        da = aa - jnp.concatenate([z0c, aa[:, :-1]], axis=1)
        db = bb - jnp.concatenate([z0c, bb[:, :-1]], axis=1)
        u_col = 0.03125 + 0.0625 * _fiota((1, N_IMP, 1), 1)
        S = (cl[:, None, :] <= u_col).astype(f32)
        A = jnp.sum(S * da[:, None, :], axis=-1)
        Bc = jnp.sum(S * db[:, None, :], axis=-1)
        new_z = A + Bc * (0.03125 + 0.0625 * _fiota((1, N_IMP), 1))
        if not last:
            new_sdf = sdf_eval(new_z)

        n_out = n + N_IMP
        iota_out = _fiota((1, n_out), 1)
        cnt = jnp.sum((z[:, None, :] <= new_z[:, :, None]).astype(f32), axis=-1)
        rnk = cnt + _fiota((1, N_IMP), 1)
        mask3 = (rnk[:, :, None] == iota_out[:, None, :]).astype(f32)
        is_new = jnp.sum(mask3, axis=1)
        newpart = jnp.sum(mask3 * new_z[:, :, None], axis=1)
        if not last:
            newsdfp = jnp.sum(mask3 * new_sdf[:, :, None], axis=1)
        rowo = _fiota((n_out, n_out), 0)
        colo = _fiota((n_out, n_out), 1)
        incl_o = (rowo <= colo).astype(f32)
        c = jnp.dot(is_new, incl_o, preferred_element_type=f32, precision=jax.lax.Precision.HIGHEST)
        zpad = jnp.concatenate([z, jnp.zeros((R, N_IMP), f32)], axis=1)
        if not last:
            spad = jnp.concatenate([sdf, jnp.zeros((R, N_IMP), f32)], axis=1)
        oldpart = jnp.zeros((R, n_out), f32)
        oldsdfp = jnp.zeros((R, n_out), f32)
        for s in range(N_IMP + 1):
            sel = (c == float(s)).astype(f32)
            if s == 0:
                zs = zpad
                ss = spad if not last else None
            else:
                pad = jnp.zeros((R, s), f32)
                zs = jnp.concatenate([pad, zpad[:, :n_out - s]], axis=1)
                if not last:
                    ss = jnp.concatenate([pad, spad[:, :n_out - s]], axis=1)
            oldpart = oldpart + sel * zs
            if not last:
                oldsdfp = oldsdfp + sel * ss
        notnew = 1.0 - is_new
        z = newpart + notnew * oldpart
        if not last:
            sdf = newsdfp + notnew * oldsdfp

    out_ref[...] = z


def kernel(rays_o, rays_d, near, far, W1, b1, W2, b2):
    f32 = jnp.float32
    B = rays_o.shape[0]
    R = R_BLOCK
    o8 = jnp.concatenate([rays_o, jnp.zeros((B, 5), f32)], axis=1)
    d8 = jnp.concatenate([rays_d, jnp.zeros((B, 5), f32)], axis=1)
    nf = jnp.concatenate([near, far, jnp.zeros((B, 6), f32)], axis=1)
    W1p = jnp.concatenate([W1, jnp.zeros((5, HID), f32)], axis=0)
    b1r = b1.reshape(1, HID)
    w2r = W2.reshape(1, HID)
    b2r = b2.reshape(1, 1)
    return pl.pallas_call(
        _ssampler_kernel,
        grid=(B // R,),
        in_specs=[
            pl.BlockSpec((R, 8), lambda i: (i, 0)),
            pl.BlockSpec((R, 8), lambda i: (i, 0)),
            pl.BlockSpec((R, 8), lambda i: (i, 0)),
            pl.BlockSpec((8, HID), lambda i: (0, 0)),
            pl.BlockSpec((1, HID), lambda i: (0, 0)),
            pl.BlockSpec((1, HID), lambda i: (0, 0)),
            pl.BlockSpec((1, 1), lambda i: (0, 0)),
        ],
        out_specs=pl.BlockSpec((R, 128), lambda i: (i, 0)),
        out_shape=jax.ShapeDtypeStruct((B, 128), f32),
    )(o8, d8, nf, W1p, b1r, w2r, b2r)


# speedup vs baseline: 1.8568x; 1.3732x over previous
import jax
import jax.numpy as jnp
from jax.experimental import pallas as pl

N_SAMPLES = 64
N_IMP = 16
STEPS = 4
HID = 256
R_BLOCK = 128


def _fiota(shape, dim):
    return jax.lax.broadcasted_iota(jnp.int32, shape, dim).astype(jnp.float32)


def _softplus(x):
    return jnp.maximum(x, 0.0) + jnp.log(1.0 + jnp.exp(-jnp.abs(x)))


def _ssampler_kernel(o_ref, d_ref, nf_ref, W1_ref, b1_ref, w2_ref, b2_ref, out_ref):
    f32 = jnp.float32
    R = o_ref.shape[0]
    o = o_ref[...]
    d = d_ref[...]
    W1 = W1_ref[...]
    b1 = b1_ref[...]
    w2 = w2_ref[...]
    b2 = b2_ref[0, 0]
    near = nf_ref[:, 0:1]
    far = nf_ref[:, 1:2]

    a_o = jnp.dot(o, W1, preferred_element_type=f32, precision=jax.lax.Precision.HIGHEST)
    a_d = jnp.dot(d, W1, preferred_element_type=f32, precision=jax.lax.Precision.HIGHEST)
    oo = jnp.sum(o * o, axis=-1, keepdims=True)
    od = jnp.sum(o * d, axis=-1, keepdims=True)
    dd = jnp.sum(d * d, axis=-1, keepdims=True)

    a_ob = (a_o + b1).reshape(R, 1, HID)
    a_d3 = a_d.reshape(R, 1, HID)
    w2b = w2.reshape(1, 1, HID)

    def sdf_eval(zv):
        h = a_ob + zv[:, :, None] * a_d3
        return jnp.sum(_softplus(h) * w2b, axis=-1) + b2

    t = _fiota((1, N_SAMPLES), 1) * (1.0 / (N_SAMPLES - 1))
    z = near * (1.0 - t) + far * t
    sdf = sdf_eval(z)

    for step in range(STEPS):
        inv_s = float(64 * 2 ** step)
        n = z.shape[1]
        m = n - 1
        last = step == STEPS - 1

        rad2 = oo + 2.0 * z * od + z * z * dd
        inside = jnp.logical_or(rad2[:, :-1] < 1.0, rad2[:, 1:] < 1.0).astype(f32)
        ps, nsdf = sdf[:, :-1], sdf[:, 1:]
        pz, nz = z[:, :-1], z[:, 1:]
        mid = (ps + nsdf) * 0.5
        dist = nz - pz
        cos = (nsdf - ps) / (dist + 1e-5)
        prev_cos = jnp.concatenate([jnp.zeros((R, 1), f32), cos[:, :-1]], axis=1)
        cos = jnp.minimum(prev_cos, cos)
        cos = jnp.clip(cos, -1000.0, 0.0) * inside
        pe = mid - cos * dist * 0.5
        ne = mid + cos * dist * 0.5
        ex = jnp.exp(jnp.clip(-pe * inv_s, -40.0, 40.0))
        ey = jnp.exp(jnp.clip(-ne * inv_s, -40.0, 40.0))
        opx = 1.0 + ex
        opy = 1.0 + ey
        alpha = ((ey - ex) + 1e-5 * opx * opy) / (opy * (1.0 + 1e-5 * opx))
        tmr = 1.0 - alpha + 1e-7
        trans = jnp.concatenate([jnp.ones((R, 1), f32), tmr[:, :-1]], axis=1)
        sh = 1
        while sh < m:
            trans = trans * jnp.concatenate(
                [jnp.ones((R, sh), f32), trans[:, :m - sh]], axis=1)
            sh *= 2
        w = alpha * trans + 1e-5
        row = _fiota((m, m), 0)
        col = _fiota((m, m), 1)
        incl_u = (row <= col).astype(f32)

        pdf = w * (1.0 / jnp.sum(w, axis=-1, keepdims=True))
        cdf_core = jnp.dot(pdf, incl_u, preferred_element_type=f32, precision=jax.lax.Precision.HIGHEST)
        cdf = jnp.concatenate([jnp.zeros((R, 1), f32), cdf_core], axis=1)
        zl, zr = z[:, :-1], z[:, 1:]
        cl, cr = cdf[:, :-1], cdf[:, 1:]
        denom = cr - cl
        denom = jnp.where(denom < 1e-5, 1.0, denom)
        bb = (zr - zl) / denom
        aa = zl - cl * bb
        z0c = jnp.zeros((R, 1), f32)
        da = aa - jnp.concatenate([z0c, aa[:, :-1]], axis=1)
        db = bb - jnp.concatenate([z0c, bb[:, :-1]], axis=1)
        u_col = 0.03125 + 0.0625 * _fiota((1, N_IMP, 1), 1)
        S = (cl[:, None, :] <= u_col).astype(f32)
        A = jnp.sum(S * da[:, None, :], axis=-1)
        Bc = jnp.sum(S * db[:, None, :], axis=-1)
        new_z = A + Bc * (0.03125 + 0.0625 * _fiota((1, N_IMP), 1))
        if not last:
            new_sdf = sdf_eval(new_z)

        n_out = n + N_IMP
        iota_out = _fiota((1, n_out), 1)
        cnt = jnp.sum((z[:, None, :] <= new_z[:, :, None]).astype(f32), axis=-1)
        rnk = cnt + _fiota((1, N_IMP), 1)
        mask3 = (rnk[:, :, None] == iota_out[:, None, :]).astype(f32)
        is_new = jnp.sum(mask3, axis=1)
        newpart = jnp.sum(mask3 * new_z[:, :, None], axis=1)
        if not last:
            newsdfp = jnp.sum(mask3 * new_sdf[:, :, None], axis=1)
        rowo = _fiota((n_out, n_out), 0)
        colo = _fiota((n_out, n_out), 1)
        incl_o = (rowo <= colo).astype(f32)
        c = jnp.dot(is_new, incl_o, preferred_element_type=f32, precision=jax.lax.Precision.HIGHEST)
        zpad = jnp.concatenate([z, jnp.zeros((R, N_IMP), f32)], axis=1)
        if not last:
            spad = jnp.concatenate([sdf, jnp.zeros((R, N_IMP), f32)], axis=1)
        oldpart = jnp.zeros((R, n_out), f32)
        oldsdfp = jnp.zeros((R, n_out), f32)
        for s in range(N_IMP + 1):
            sel = (c == float(s)).astype(f32)
            if s == 0:
                zs = zpad
                ss = spad if not last else None
            else:
                pad = jnp.zeros((R, s), f32)
                zs = jnp.concatenate([pad, zpad[:, :n_out - s]], axis=1)
                if not last:
                    ss = jnp.concatenate([pad, spad[:, :n_out - s]], axis=1)
            oldpart = oldpart + sel * zs
            if not last:
                oldsdfp = oldsdfp + sel * ss
        notnew = 1.0 - is_new
        z = newpart + notnew * oldpart
        if not last:
            sdf = newsdfp + notnew * oldsdfp

    out_ref[...] = z


def kernel(rays_o, rays_d, near, far, W1, b1, W2, b2):
    f32 = jnp.float32
    B = rays_o.shape[0]
    R = R_BLOCK
    o8 = jnp.concatenate([rays_o, jnp.zeros((B, 5), f32)], axis=1)
    d8 = jnp.concatenate([rays_d, jnp.zeros((B, 5), f32)], axis=1)
    nf = jnp.concatenate([near, far, jnp.zeros((B, 6), f32)], axis=1)
    W1p = jnp.concatenate([W1, jnp.zeros((5, HID), f32)], axis=0)
    b1r = b1.reshape(1, HID)
    w2r = W2.reshape(1, HID)
    b2r = b2.reshape(1, 1)
    return pl.pallas_call(
        _ssampler_kernel,
        grid=(B // R,),
        in_specs=[
            pl.BlockSpec((R, 8), lambda i: (i, 0)),
            pl.BlockSpec((R, 8), lambda i: (i, 0)),
            pl.BlockSpec((R, 8), lambda i: (i, 0)),
            pl.BlockSpec((8, HID), lambda i: (0, 0)),
            pl.BlockSpec((1, HID), lambda i: (0, 0)),
            pl.BlockSpec((1, HID), lambda i: (0, 0)),
            pl.BlockSpec((1, 1), lambda i: (0, 0)),
        ],
        out_specs=pl.BlockSpec((R, 128), lambda i: (i, 0)),
        out_shape=jax.ShapeDtypeStruct((B, 128), f32),
    )(o8, d8, nf, W1p, b1r, w2r, b2r)
